# Initial kernel scaffold; baseline (speedup 1.0000x reference)
import functools

import jax
import jax.numpy as jnp
from jax import lax
from jax.experimental import pallas as pl
from jax.experimental.pallas import tpu as pltpu
from jax.experimental.pallas import tpu_sc as plsc

N = 10000
E = 10000
D = 128
NNZ = 320000
NP = 10240
TRASH = 10000
NC, NS = 2, 16
NW = NC * NS
BLK = 128
EP = 327680
BPW = EP // (NW * BLK)
RPT = NP // NS


def _matmul_body(x_ref, w_ref, o_ref):
    o_ref[...] = jnp.dot(x_ref[...], w_ref[...],
                         preferred_element_type=jnp.float32)


def _matmul(x_pad, theta):
    return pl.pallas_call(
        _matmul_body,
        grid=(NP // 1024,),
        in_specs=[
            pl.BlockSpec((1024, D), lambda i: (i, 0)),
            pl.BlockSpec((D, D), lambda i: (0, 0)),
        ],
        out_specs=pl.BlockSpec((1024, D), lambda i: (i, 0)),
        out_shape=jax.ShapeDtypeStruct((NP, D), jnp.float32),
    )(x_pad, theta)


def _combine_body(s_ref, d_ref, b_ref, o_ref):
    deg = d_ref[0, :, 0] + d_ref[1, :, 0]
    inv = jnp.where(deg > 0, 1.0 / deg, 0.0)
    o_ref[...] = (s_ref[0] + s_ref[1]) * inv[:, None] + b_ref[...]


def _combine(s_partial, deg_partial, bias_row):
    R = 1024
    return pl.pallas_call(
        _combine_body,
        grid=(NP // R,),
        in_specs=[
            pl.BlockSpec((2, R, D), lambda i: (0, i, 0)),
            pl.BlockSpec((2, R, 16), lambda i: (0, i, 0)),
            pl.BlockSpec((1, D), lambda i: (0, 0)),
        ],
        out_specs=pl.BlockSpec((R, D), lambda i: (i, 0)),
        out_shape=jax.ShapeDtypeStruct((NP, D), jnp.float32),
    )(s_partial, deg_partial, bias_row)


def _make_sc_scatter(want_deg: bool):
    mesh = plsc.VectorSubcoreMesh(core_axis_name="c", subcore_axis_name="s",
                                  num_cores=NC, num_subcores=NS)
    out_type = [jax.ShapeDtypeStruct((NC, NP, D), jnp.float32)]
    scratch = [
        pltpu.VMEM((BPW, BLK), jnp.int32),
        pltpu.VMEM((BPW, BLK), jnp.int32),
        pltpu.VMEM((BLK, D), jnp.float32),
        pltpu.VMEM_SHARED((NP, D), jnp.float32),
        pltpu.SemaphoreType.DMA,
    ]
    if want_deg:
        out_type += [jax.ShapeDtypeStruct((NC, NP, 16), jnp.float32),
                     jax.ShapeDtypeStruct((NC, NP, 16), jnp.float32)]
        scratch += [
            pltpu.VMEM((BLK, 16), jnp.float32),
            pltpu.VMEM((RPT, 16), jnp.float32),
            pltpu.VMEM_SHARED((NP, 16), jnp.float32),
            pltpu.VMEM_SHARED((NP, 16), jnp.float32),
        ]

    def body(tab, gidx, sidx, *refs):
        if want_deg:
            (s_out, dS_out, dG_out, gidx_v, sidx_v, rows_v, acc_sh, gsem,
             ones_v, dstage_v, dS_sh, dG_sh) = refs
        else:
            s_out, gidx_v, sidx_v, rows_v, acc_sh, gsem = refs
        c = lax.axis_index("c")
        s = lax.axis_index("s")
        wid = c * NS + s
        base = s * RPT

        def zrow(i, _):
            def zlane(j, carry):
                rows_v[i, pl.ds(j * 16, 16)] = jnp.zeros((16,), jnp.float32)
                return carry
            return lax.fori_loop(0, D // 16, zlane, 0)
        lax.fori_loop(0, BLK, zrow, 0)
        for k in range(RPT // BLK):
            pltpu.sync_copy(rows_v, acc_sh.at[pl.ds(base + k * BLK, BLK)])
        if want_deg:
            def zdeg(i, carry):
                dstage_v[i] = jnp.zeros((16,), jnp.float32)
                return carry
            lax.fori_loop(0, RPT, zdeg, 0)
            pltpu.sync_copy(dstage_v, dS_sh.at[pl.ds(base, RPT)])
            pltpu.sync_copy(dstage_v, dG_sh.at[pl.ds(base, RPT)])

            def onesrow(i, carry):
                ones_v[i] = jnp.ones((16,), jnp.float32)
                return carry
            lax.fori_loop(0, BLK, onesrow, 0)
        plsc.subcore_barrier()

        pltpu.sync_copy(gidx.at[wid], gidx_v)
        pltpu.sync_copy(sidx.at[wid], sidx_v)

        def blk_step(j, carry):
            pltpu.async_copy(tab.at[gidx_v.at[j]], rows_v, gsem).wait()
            pltpu.sync_copy(rows_v, acc_sh.at[sidx_v.at[j]], add=True)
            if want_deg:
                pltpu.sync_copy(ones_v, dS_sh.at[sidx_v.at[j]], add=True)
                pltpu.sync_copy(ones_v, dG_sh.at[gidx_v.at[j]], add=True)
            return carry
        lax.fori_loop(0, BPW, blk_step, 0)
        plsc.subcore_barrier()

        for k in range(RPT // BLK):
            r0 = base + k * BLK
            pltpu.sync_copy(acc_sh.at[pl.ds(r0, BLK)], rows_v)
            pltpu.sync_copy(rows_v, s_out.at[c].at[pl.ds(r0, BLK)])
        if want_deg:
            pltpu.sync_copy(dS_sh.at[pl.ds(base, RPT)], dstage_v)
            pltpu.sync_copy(dstage_v, dS_out.at[c].at[pl.ds(base, RPT)])
            pltpu.sync_copy(dG_sh.at[pl.ds(base, RPT)], dstage_v)
            pltpu.sync_copy(dstage_v, dG_out.at[c].at[pl.ds(base, RPT)])

    return pl.kernel(body, out_type=out_type, mesh=mesh,
                     scratch_types=scratch)


_sc_scatter_deg = _make_sc_scatter(True)
_sc_scatter = _make_sc_scatter(False)


def kernel(x, H, theta, bias):
    x = x.astype(jnp.float32)
    nidx = H[0].astype(jnp.int32)
    eidx = H[1].astype(jnp.int32)
    pad = jnp.full((EP - NNZ,), TRASH, jnp.int32)
    nidx_p = jnp.concatenate([nidx, pad]).reshape(NW, BPW, BLK)
    eidx_p = jnp.concatenate([eidx, pad]).reshape(NW, BPW, BLK)
    x_pad = jnp.concatenate([x, jnp.zeros((NP - N, D), jnp.float32)])

    xw = _matmul(x_pad, theta)
    s1p, deg_e, deg_n = _sc_scatter_deg(xw, nidx_p, eidx_p)
    zeros_row = jnp.zeros((1, D), jnp.float32)
    hyedge_ft = _combine(s1p, deg_e, zeros_row)
    s2p, = _sc_scatter(hyedge_ft, eidx_p, nidx_p)
    node_ft = _combine(s2p, deg_n, bias.reshape(1, D))
    return node_ft[:N]


# speedup vs baseline: 5.6057x; 5.6057x over previous
import jax
import jax.numpy as jnp
from jax import lax
from jax.experimental import pallas as pl
from jax.experimental.pallas import tpu as pltpu
from jax.experimental.pallas import tpu_sc as plsc

N = 10000
E = 10000
D = 128
NNZ = 320000
NP = 10240
TRASH = 10000
NC, NS = 2, 16
NW = NC * NS
BLK = 128
EP = 327680
BPW = EP // (NW * BLK)
RPT = NP // NS

_MESH = plsc.VectorSubcoreMesh(core_axis_name="c", subcore_axis_name="s",
                               num_cores=NC, num_subcores=NS)


def _matmul_body(x_ref, w_ref, o_ref):
    o_ref[...] = jnp.dot(x_ref[...], w_ref[...],
                         preferred_element_type=jnp.float32)


def _matmul(x_pad, theta):
    return pl.pallas_call(
        _matmul_body,
        grid=(NP // 1024,),
        in_specs=[
            pl.BlockSpec((1024, D), lambda i: (i, 0)),
            pl.BlockSpec((D, D), lambda i: (0, 0)),
        ],
        out_specs=pl.BlockSpec((1024, D), lambda i: (i, 0)),
        out_shape=jax.ShapeDtypeStruct((NP, D), jnp.float32),
    )(x_pad, theta)


def _combine_body(s_ref, d_ref, b_ref, o_ref):
    deg = jnp.sum(d_ref[...], axis=0)
    inv = jnp.where(deg > 0, 1.0 / deg, 0.0)
    o_ref[...] = (s_ref[0] + s_ref[1]) * inv[:, None] + b_ref[...]


def _combine(s_partial, deg_partial, bias_row):
    R = 1024
    return pl.pallas_call(
        _combine_body,
        grid=(NP // R,),
        in_specs=[
            pl.BlockSpec((2, R, D), lambda i: (0, i, 0)),
            pl.BlockSpec((NW, R), lambda i: (0, i)),
            pl.BlockSpec((1, D), lambda i: (0, 0)),
        ],
        out_specs=pl.BlockSpec((R, D), lambda i: (i, 0)),
        out_shape=jax.ShapeDtypeStruct((NP, D), jnp.float32),
    )(s_partial, deg_partial, bias_row)


def _sc_scatter_body(tab, gidx, sidx, s_out, gidx_v, sidx_v, rows_v, acc_sh,
                     gsem):
    c = lax.axis_index("c")
    s = lax.axis_index("s")
    wid = c * NS + s
    base = s * RPT

    def zrow(i, carry):
        def zlane(j, carry2):
            rows_v[i, pl.ds(j * 16, 16)] = jnp.zeros((16,), jnp.float32)
            return carry2
        return lax.fori_loop(0, D // 16, zlane, carry)
    lax.fori_loop(0, BLK, zrow, 0)
    for k in range(RPT // BLK):
        pltpu.sync_copy(rows_v, acc_sh.at[pl.ds(base + k * BLK, BLK)])
    plsc.subcore_barrier()

    pltpu.sync_copy(gidx.at[wid], gidx_v)
    pltpu.sync_copy(sidx.at[wid], sidx_v)

    def blk_step(j, carry):
        pltpu.async_copy(tab.at[gidx_v.at[j]], rows_v, gsem).wait()
        pltpu.sync_copy(rows_v, acc_sh.at[sidx_v.at[j]], add=True)
        return carry
    lax.fori_loop(0, BPW, blk_step, 0)
    plsc.subcore_barrier()

    for k in range(RPT // BLK):
        r0 = base + k * BLK
        pltpu.sync_copy(acc_sh.at[pl.ds(r0, BLK)], rows_v)
        pltpu.sync_copy(rows_v, s_out.at[c].at[pl.ds(r0, BLK)])


_sc_scatter = pl.kernel(
    _sc_scatter_body,
    out_type=[jax.ShapeDtypeStruct((NC, NP, D), jnp.float32)],
    mesh=_MESH,
    scratch_types=[
        pltpu.VMEM((BPW, BLK), jnp.int32),
        pltpu.VMEM((BPW, BLK), jnp.int32),
        pltpu.VMEM((BLK, D), jnp.float32),
        pltpu.VMEM_SHARED((NP, D), jnp.float32),
        pltpu.SemaphoreType.DMA,
    ],
)


_CHUNK = EP // NW


def _sc_degrees_body(nidx, eidx, dN_out, dE_out, nidx_v, eidx_v, dN_v, dE_v):
    c = lax.axis_index("c")
    s = lax.axis_index("s")
    wid = c * NS + s

    def z(i, carry):
        dN_v[pl.ds(i * 16, 16)] = jnp.zeros((16,), jnp.float32)
        dE_v[pl.ds(i * 16, 16)] = jnp.zeros((16,), jnp.float32)
        return carry
    lax.fori_loop(0, NP // 16, z, 0)

    pltpu.sync_copy(nidx.at[wid], nidx_v)
    pltpu.sync_copy(eidx.at[wid], eidx_v)

    ones16 = jnp.ones((16,), jnp.float32)

    def step(j, carry):
        nv = nidx_v[pl.ds(j * 16, 16)]
        ev = eidx_v[pl.ds(j * 16, 16)]
        plsc.addupdate_scatter(dN_v, [nv], ones16)
        plsc.addupdate_scatter(dE_v, [ev], ones16)
        return carry
    lax.fori_loop(0, _CHUNK // 16, step, 0)

    pltpu.sync_copy(dN_v, dN_out.at[wid])
    pltpu.sync_copy(dE_v, dE_out.at[wid])


_sc_degrees = pl.kernel(
    _sc_degrees_body,
    out_type=[jax.ShapeDtypeStruct((NW, NP), jnp.float32),
              jax.ShapeDtypeStruct((NW, NP), jnp.float32)],
    mesh=_MESH,
    scratch_types=[
        pltpu.VMEM((_CHUNK,), jnp.int32),
        pltpu.VMEM((_CHUNK,), jnp.int32),
        pltpu.VMEM((NP,), jnp.float32),
        pltpu.VMEM((NP,), jnp.float32),
    ],
    compiler_params=pltpu.CompilerParams(needs_layout_passes=False),
)


def kernel(x, H, theta, bias):
    x = x.astype(jnp.float32)
    nidx = H[0].astype(jnp.int32)
    eidx = H[1].astype(jnp.int32)
    pad = jnp.full((EP - NNZ,), TRASH, jnp.int32)
    nidx_f = jnp.concatenate([nidx, pad])
    eidx_f = jnp.concatenate([eidx, pad])
    nidx_p = nidx_f.reshape(NW, BPW, BLK)
    eidx_p = eidx_f.reshape(NW, BPW, BLK)
    x_pad = jnp.concatenate([x, jnp.zeros((NP - N, D), jnp.float32)])

    deg_n, deg_e = _sc_degrees(nidx_f.reshape(NW, _CHUNK),
                               eidx_f.reshape(NW, _CHUNK))
    xw = _matmul(x_pad, theta)
    (s1p,) = _sc_scatter(xw, nidx_p, eidx_p)
    zeros_row = jnp.zeros((1, D), jnp.float32)
    hyedge_ft = _combine(s1p, deg_e, zeros_row)
    (s2p,) = _sc_scatter(hyedge_ft, eidx_p, nidx_p)
    node_ft = _combine(s2p, deg_n, bias.reshape(1, D))
    return node_ft[:N]


# speedup vs baseline: 6.0277x; 1.0753x over previous
import jax
import jax.numpy as jnp
from jax import lax
from jax.experimental import pallas as pl
from jax.experimental.pallas import tpu as pltpu
from jax.experimental.pallas import tpu_sc as plsc

N = 10000
E = 10000
D = 128
NNZ = 320000
NP = 10240
TRASH = 10000
NC, NS = 2, 16
NW = NC * NS
BLK = 128
EP = 327680
BPW = EP // (NW * BLK)
RPT = NP // NS

_MESH = plsc.VectorSubcoreMesh(core_axis_name="c", subcore_axis_name="s",
                               num_cores=NC, num_subcores=NS)


def _matmul_body(x_ref, w_ref, o_ref):
    o_ref[...] = jnp.dot(x_ref[...], w_ref[...],
                         preferred_element_type=jnp.float32)


def _matmul(x_pad, theta):
    return pl.pallas_call(
        _matmul_body,
        grid=(NP // 1024,),
        in_specs=[
            pl.BlockSpec((1024, D), lambda i: (i, 0)),
            pl.BlockSpec((D, D), lambda i: (0, 0)),
        ],
        out_specs=pl.BlockSpec((1024, D), lambda i: (i, 0)),
        out_shape=jax.ShapeDtypeStruct((NP, D), jnp.float32),
    )(x_pad, theta)


def _combine_body(s_ref, d_ref, b_ref, o_ref):
    deg = jnp.sum(d_ref[...], axis=0)
    inv = jnp.where(deg > 0, 1.0 / deg, 0.0)
    o_ref[...] = (s_ref[0] + s_ref[1]) * inv[:, None] + b_ref[...]


def _combine(s_partial, deg_partial, bias_row):
    R = 1024
    return pl.pallas_call(
        _combine_body,
        grid=(NP // R,),
        in_specs=[
            pl.BlockSpec((2, R, D), lambda i: (0, i, 0)),
            pl.BlockSpec((NW, R), lambda i: (0, i)),
            pl.BlockSpec((1, D), lambda i: (0, 0)),
        ],
        out_specs=pl.BlockSpec((R, D), lambda i: (i, 0)),
        out_shape=jax.ShapeDtypeStruct((NP, D), jnp.float32),
    )(s_partial, deg_partial, bias_row)


HB = BPW // 2


def _sc_scatter_body(tab, gidx, sidx, s_out, gidx_v, sidx_v, rows_a, rows_b,
                     acc_sh, gsem_a, gsem_b):
    c = lax.axis_index("c")
    s = lax.axis_index("s")
    wid = c * NS + s
    base = s * RPT

    def zrow(i, carry):
        def zlane(j, carry2):
            rows_a[i, pl.ds(j * 16, 16)] = jnp.zeros((16,), jnp.float32)
            return carry2
        return lax.fori_loop(0, D // 16, zlane, carry)
    lax.fori_loop(0, BLK, zrow, 0)
    for k in range(RPT // BLK):
        pltpu.sync_copy(rows_a, acc_sh.at[pl.ds(base + k * BLK, BLK)])
    plsc.subcore_barrier()

    for h in range(2):
        pltpu.sync_copy(gidx.at[wid].at[pl.ds(h * HB, HB)], gidx_v)
        pltpu.sync_copy(sidx.at[wid].at[pl.ds(h * HB, HB)], sidx_v)
        pltpu.async_copy(tab.at[gidx_v.at[0]], rows_a, gsem_a)

        def pair(i, carry):
            j = 2 * i
            pltpu.make_async_copy(tab.at[gidx_v.at[j]], rows_a, gsem_a).wait()
            pltpu.async_copy(tab.at[gidx_v.at[j + 1]], rows_b, gsem_b)
            pltpu.sync_copy(rows_a, acc_sh.at[sidx_v.at[j]], add=True)
            pltpu.make_async_copy(tab.at[gidx_v.at[j + 1]], rows_b,
                                  gsem_b).wait()

            @pl.when(j + 2 < HB)
            def _():
                pltpu.async_copy(tab.at[gidx_v.at[j + 2]], rows_a, gsem_a)
            pltpu.sync_copy(rows_b, acc_sh.at[sidx_v.at[j + 1]], add=True)
            return carry
        lax.fori_loop(0, HB // 2, pair, 0)
    plsc.subcore_barrier()

    for k in range(RPT // BLK):
        r0 = base + k * BLK
        pltpu.sync_copy(acc_sh.at[pl.ds(r0, BLK)], rows_a)
        pltpu.sync_copy(rows_a, s_out.at[c].at[pl.ds(r0, BLK)])


_sc_scatter = pl.kernel(
    _sc_scatter_body,
    out_type=[jax.ShapeDtypeStruct((NC, NP, D), jnp.float32)],
    mesh=_MESH,
    scratch_types=[
        pltpu.VMEM((HB, BLK), jnp.int32),
        pltpu.VMEM((HB, BLK), jnp.int32),
        pltpu.VMEM((BLK, D), jnp.float32),
        pltpu.VMEM((BLK, D), jnp.float32),
        pltpu.VMEM_SHARED((NP, D), jnp.float32),
        pltpu.SemaphoreType.DMA,
        pltpu.SemaphoreType.DMA,
    ],
)


_CHUNK = EP // NW


def _sc_degrees_body(nidx, eidx, dN_out, dE_out, nidx_v, eidx_v, dN_v, dE_v):
    c = lax.axis_index("c")
    s = lax.axis_index("s")
    wid = c * NS + s

    def z(i, carry):
        dN_v[pl.ds(i * 16, 16)] = jnp.zeros((16,), jnp.float32)
        dE_v[pl.ds(i * 16, 16)] = jnp.zeros((16,), jnp.float32)
        return carry
    lax.fori_loop(0, NP // 16, z, 0)

    pltpu.sync_copy(nidx.at[wid], nidx_v)
    pltpu.sync_copy(eidx.at[wid], eidx_v)

    ones16 = jnp.ones((16,), jnp.float32)

    def step(j, carry):
        nv = nidx_v[pl.ds(j * 16, 16)]
        ev = eidx_v[pl.ds(j * 16, 16)]
        plsc.addupdate_scatter(dN_v, [nv], ones16)
        plsc.addupdate_scatter(dE_v, [ev], ones16)
        return carry
    lax.fori_loop(0, _CHUNK // 16, step, 0)

    pltpu.sync_copy(dN_v, dN_out.at[wid])
    pltpu.sync_copy(dE_v, dE_out.at[wid])


_sc_degrees = pl.kernel(
    _sc_degrees_body,
    out_type=[jax.ShapeDtypeStruct((NW, NP), jnp.float32),
              jax.ShapeDtypeStruct((NW, NP), jnp.float32)],
    mesh=_MESH,
    scratch_types=[
        pltpu.VMEM((_CHUNK,), jnp.int32),
        pltpu.VMEM((_CHUNK,), jnp.int32),
        pltpu.VMEM((NP,), jnp.float32),
        pltpu.VMEM((NP,), jnp.float32),
    ],
    compiler_params=pltpu.CompilerParams(needs_layout_passes=False),
)


def kernel(x, H, theta, bias):
    x = x.astype(jnp.float32)
    nidx = H[0].astype(jnp.int32)
    eidx = H[1].astype(jnp.int32)
    pad = jnp.full((EP - NNZ,), TRASH, jnp.int32)
    nidx_f = jnp.concatenate([nidx, pad])
    eidx_f = jnp.concatenate([eidx, pad])
    nidx_p = nidx_f.reshape(NW, BPW, BLK)
    eidx_p = eidx_f.reshape(NW, BPW, BLK)
    x_pad = jnp.concatenate([x, jnp.zeros((NP - N, D), jnp.float32)])

    deg_n, deg_e = _sc_degrees(nidx_f.reshape(NW, _CHUNK),
                               eidx_f.reshape(NW, _CHUNK))
    xw = _matmul(x_pad, theta)
    (s1p,) = _sc_scatter(xw, nidx_p, eidx_p)
    zeros_row = jnp.zeros((1, D), jnp.float32)
    hyedge_ft = _combine(s1p, deg_e, zeros_row)
    (s2p,) = _sc_scatter(hyedge_ft, eidx_p, nidx_p)
    node_ft = _combine(s2p, deg_n, bias.reshape(1, D))
    return node_ft[:N]


# speedup vs baseline: 6.0483x; 1.0034x over previous
import jax
import jax.numpy as jnp
from jax import lax
from jax.experimental import pallas as pl
from jax.experimental.pallas import tpu as pltpu
from jax.experimental.pallas import tpu_sc as plsc

N = 10000
E = 10000
D = 128
NNZ = 320000
NP = 10240
TRASH = 10000
NC, NS = 2, 16
NW = NC * NS
BLK = 128
EP = 327680
BPW = EP // (NW * BLK)
RPT = NP // NS

_MESH = plsc.VectorSubcoreMesh(core_axis_name="c", subcore_axis_name="s",
                               num_cores=NC, num_subcores=NS)


def _matmul_body(x_ref, w_ref, o_ref):
    o_ref[...] = jnp.dot(x_ref[...], w_ref[...],
                         preferred_element_type=jnp.float32)


def _matmul(x_pad, theta):
    return pl.pallas_call(
        _matmul_body,
        grid=(NP // 1024,),
        in_specs=[
            pl.BlockSpec((1024, D), lambda i: (i, 0)),
            pl.BlockSpec((D, D), lambda i: (0, 0)),
        ],
        out_specs=pl.BlockSpec((1024, D), lambda i: (i, 0)),
        out_shape=jax.ShapeDtypeStruct((NP, D), jnp.float32),
    )(x_pad, theta)


def _combine_body(s_ref, d_ref, b_ref, o_ref):
    deg = jnp.sum(d_ref[...], axis=0)
    inv = jnp.where(deg > 0, 1.0 / deg, 0.0)
    o_ref[...] = (s_ref[0] + s_ref[1]) * inv[:, None] + b_ref[...]


def _combine(s_partial, deg_partial, bias_row):
    R = 1024
    return pl.pallas_call(
        _combine_body,
        grid=(NP // R,),
        in_specs=[
            pl.BlockSpec((2, R, D), lambda i: (0, i, 0)),
            pl.BlockSpec((NW, R), lambda i: (0, i)),
            pl.BlockSpec((1, D), lambda i: (0, 0)),
        ],
        out_specs=pl.BlockSpec((R, D), lambda i: (i, 0)),
        out_shape=jax.ShapeDtypeStruct((NP, D), jnp.float32),
    )(s_partial, deg_partial, bias_row)


HB = BPW // 2


def _sc_scatter_body(tab, gidx, sidx, s_out, gidx_v, sidx_v, rows_a, rows_b,
                     acc_sh, gsem_a, gsem_b):
    c = lax.axis_index("c")
    s = lax.axis_index("s")
    wid = c * NS + s
    base = s * RPT

    def zrow(i, carry):
        def zlane(j, carry2):
            rows_a[i, pl.ds(j * 16, 16)] = jnp.zeros((16,), jnp.float32)
            return carry2
        return lax.fori_loop(0, D // 16, zlane, carry)
    lax.fori_loop(0, BLK, zrow, 0)
    for k in range(RPT // BLK):
        pltpu.sync_copy(rows_a, acc_sh.at[pl.ds(base + k * BLK, BLK)])
    plsc.subcore_barrier()

    for h in range(2):
        pltpu.sync_copy(gidx.at[wid].at[pl.ds(h * HB, HB)], gidx_v)
        pltpu.sync_copy(sidx.at[wid].at[pl.ds(h * HB, HB)], sidx_v)
        def gat(j, buf, sem):
            for k in range(4):
                pltpu.async_copy(
                    tab.at[gidx_v.at[j].at[pl.ds(32 * k, 32)]],
                    buf.at[pl.ds(32 * k, 32)], sem)

        gat(0, rows_a, gsem_a)

        def pair(i, carry):
            j = 2 * i
            pltpu.make_async_copy(tab.at[gidx_v.at[j]], rows_a, gsem_a).wait()
            gat(j + 1, rows_b, gsem_b)
            pltpu.make_async_copy(tab.at[gidx_v.at[j + 1]], rows_b,
                                  gsem_b).wait()

            @pl.when(j + 2 < HB)
            def _():
                gat(j + 2, rows_a, gsem_a)
            return carry
        lax.fori_loop(0, HB // 2, pair, 0)
    plsc.subcore_barrier()

    for k in range(RPT // BLK):
        r0 = base + k * BLK
        pltpu.sync_copy(acc_sh.at[pl.ds(r0, BLK)], rows_a)
        pltpu.sync_copy(rows_a, s_out.at[c].at[pl.ds(r0, BLK)])


_sc_scatter = pl.kernel(
    _sc_scatter_body,
    out_type=[jax.ShapeDtypeStruct((NC, NP, D), jnp.float32)],
    mesh=_MESH,
    scratch_types=[
        pltpu.VMEM((HB, BLK), jnp.int32),
        pltpu.VMEM((HB, BLK), jnp.int32),
        pltpu.VMEM((BLK, D), jnp.float32),
        pltpu.VMEM((BLK, D), jnp.float32),
        pltpu.VMEM_SHARED((NP, D), jnp.float32),
        pltpu.SemaphoreType.DMA,
        pltpu.SemaphoreType.DMA,
    ],
)


_CHUNK = EP // NW


def _sc_degrees_body(nidx, eidx, dN_out, dE_out, nidx_v, eidx_v, dN_v, dE_v):
    c = lax.axis_index("c")
    s = lax.axis_index("s")
    wid = c * NS + s

    def z(i, carry):
        dN_v[pl.ds(i * 16, 16)] = jnp.zeros((16,), jnp.float32)
        dE_v[pl.ds(i * 16, 16)] = jnp.zeros((16,), jnp.float32)
        return carry
    lax.fori_loop(0, NP // 16, z, 0)

    pltpu.sync_copy(nidx.at[wid], nidx_v)
    pltpu.sync_copy(eidx.at[wid], eidx_v)

    ones16 = jnp.ones((16,), jnp.float32)

    def step(j, carry):
        nv = nidx_v[pl.ds(j * 16, 16)]
        ev = eidx_v[pl.ds(j * 16, 16)]
        plsc.addupdate_scatter(dN_v, [nv], ones16)
        plsc.addupdate_scatter(dE_v, [ev], ones16)
        return carry
    lax.fori_loop(0, _CHUNK // 16, step, 0)

    pltpu.sync_copy(dN_v, dN_out.at[wid])
    pltpu.sync_copy(dE_v, dE_out.at[wid])


_sc_degrees = pl.kernel(
    _sc_degrees_body,
    out_type=[jax.ShapeDtypeStruct((NW, NP), jnp.float32),
              jax.ShapeDtypeStruct((NW, NP), jnp.float32)],
    mesh=_MESH,
    scratch_types=[
        pltpu.VMEM((_CHUNK,), jnp.int32),
        pltpu.VMEM((_CHUNK,), jnp.int32),
        pltpu.VMEM((NP,), jnp.float32),
        pltpu.VMEM((NP,), jnp.float32),
    ],
    compiler_params=pltpu.CompilerParams(needs_layout_passes=False),
)


def kernel(x, H, theta, bias):
    x = x.astype(jnp.float32)
    nidx = H[0].astype(jnp.int32)
    eidx = H[1].astype(jnp.int32)
    pad = jnp.full((EP - NNZ,), TRASH, jnp.int32)
    nidx_f = jnp.concatenate([nidx, pad])
    eidx_f = jnp.concatenate([eidx, pad])
    nidx_p = nidx_f.reshape(NW, BPW, BLK)
    eidx_p = eidx_f.reshape(NW, BPW, BLK)
    x_pad = jnp.concatenate([x, jnp.zeros((NP - N, D), jnp.float32)])

    deg_n, deg_e = _sc_degrees(nidx_f.reshape(NW, _CHUNK),
                               eidx_f.reshape(NW, _CHUNK))
    xw = _matmul(x_pad, theta)
    (s1p,) = _sc_scatter(xw, nidx_p, eidx_p)
    zeros_row = jnp.zeros((1, D), jnp.float32)
    hyedge_ft = _combine(s1p, deg_e, zeros_row)
    (s2p,) = _sc_scatter(hyedge_ft, eidx_p, nidx_p)
    node_ft = _combine(s2p, deg_n, bias.reshape(1, D))
    return node_ft[:N]


# speedup vs baseline: 16.0982x; 2.6616x over previous
import jax
import jax.numpy as jnp
from jax import lax
from jax.experimental import pallas as pl
from jax.experimental.pallas import tpu as pltpu
from jax.experimental.pallas import tpu_sc as plsc

N = 10000
E = 10000
D = 128
NNZ = 320000
NP = 10240
NC, NS = 2, 16
NW = NC * NS
CHUNK = NNZ // NW
HALF = 5120
ACC = 5128
TROW = 10008
GDUM = 10000
SDUM = HALF
CAP = 10240
BLK = 32

_MESH = plsc.VectorSubcoreMesh(core_axis_name="c", subcore_axis_name="s",
                               num_cores=NC, num_subcores=NS)


def _matmul_body(x_ref, w_ref, o_ref):
    o_ref[...] = jnp.dot(x_ref[...], w_ref[...],
                         preferred_element_type=jnp.float32)


def _matmul(x_pad, theta):
    return pl.pallas_call(
        _matmul_body,
        grid=(NP // 1024,),
        in_specs=[
            pl.BlockSpec((1024, D), lambda i: (i, 0)),
            pl.BlockSpec((D, D), lambda i: (0, 0)),
        ],
        out_specs=pl.BlockSpec((1024, D), lambda i: (i, 0)),
        out_shape=jax.ShapeDtypeStruct((NP, D), jnp.float32),
    )(x_pad, theta)


def _combine_body(s_ref, d_ref, b_ref, o_ref):
    deg = jnp.sum(d_ref[...], axis=0)
    inv = jnp.where(deg > 0, 1.0 / deg, 0.0)
    o_ref[...] = s_ref[...] * inv[:, None] + b_ref[...]


def _combine(s_sum, deg_partial, bias_row):
    R = 1024
    return pl.pallas_call(
        _combine_body,
        grid=(NP // R,),
        in_specs=[
            pl.BlockSpec((R, D), lambda i: (i, 0)),
            pl.BlockSpec((NW, R), lambda i: (0, i)),
            pl.BlockSpec((1, D), lambda i: (0, 0)),
        ],
        out_specs=pl.BlockSpec((R, D), lambda i: (i, 0)),
        out_shape=jax.ShapeDtypeStruct((NP, D), jnp.float32),
    )(s_sum, deg_partial, bias_row)


def _sc_scan_body(nidx, eidx, dN_out, dE_out, b1g_o, b1s_o, b2g_o, b2s_o,
                  cnt_o, nidx_v, eidx_v, dN_v, dE_v, b1g0, b1s0, b1g1, b1s1,
                  b2g0, b2s0, b2g1, b2s1, cnt_v):
    c = lax.axis_index("c")
    s = lax.axis_index("s")
    wid = c * NS + s

    def z(i, carry):
        dN_v[pl.ds(i * 16, 16)] = jnp.zeros((16,), jnp.float32)
        dE_v[pl.ds(i * 16, 16)] = jnp.zeros((16,), jnp.float32)
        return carry
    lax.fori_loop(0, NP // 16, z, 0)

    pltpu.sync_copy(nidx.at[wid], nidx_v)
    pltpu.sync_copy(eidx.at[wid], eidx_v)

    ones16 = jnp.ones((16,), jnp.float32)

    def step(j, carry):
        c10, c11, c20, c21 = carry
        nv = nidx_v[pl.ds(j * 16, 16)]
        ev = eidx_v[pl.ds(j * 16, 16)]
        plsc.addupdate_scatter(dN_v, [nv], ones16)
        plsc.addupdate_scatter(dE_v, [ev], ones16)
        m = ev < HALF
        plsc.store_compressed(b1g0.at[pl.ds(c10, 16)], nv, mask=m)
        plsc.store_compressed(b1s0.at[pl.ds(c10, 16)], ev, mask=m)
        pc1 = jnp.max(plsc.all_reduce_population_count(m))
        mn = jnp.logical_not(m)
        plsc.store_compressed(b1g1.at[pl.ds(c11, 16)], nv, mask=mn)
        plsc.store_compressed(b1s1.at[pl.ds(c11, 16)], ev - HALF, mask=mn)
        m2 = nv < HALF
        plsc.store_compressed(b2g0.at[pl.ds(c20, 16)], ev, mask=m2)
        plsc.store_compressed(b2s0.at[pl.ds(c20, 16)], nv, mask=m2)
        pc2 = jnp.max(plsc.all_reduce_population_count(m2))
        m2n = jnp.logical_not(m2)
        plsc.store_compressed(b2g1.at[pl.ds(c21, 16)], ev, mask=m2n)
        plsc.store_compressed(b2s1.at[pl.ds(c21, 16)], nv - HALF, mask=m2n)
        return (c10 + pc1, c11 + (16 - pc1), c20 + pc2, c21 + (16 - pc2))

    z32 = jnp.int32(0)
    c10, c11, c20, c21 = lax.fori_loop(0, CHUNK // 16, step,
                                       (z32, z32, z32, z32))

    gdum = jnp.full((16,), GDUM, jnp.int32)
    sdum = jnp.full((16,), SDUM, jnp.int32)
    for t in range(9):
        b1g0[pl.ds(c10 + 16 * t, 16)] = gdum
        b1s0[pl.ds(c10 + 16 * t, 16)] = sdum
        b1g1[pl.ds(c11 + 16 * t, 16)] = gdum
        b1s1[pl.ds(c11 + 16 * t, 16)] = sdum
        b2g0[pl.ds(c20 + 16 * t, 16)] = gdum
        b2s0[pl.ds(c20 + 16 * t, 16)] = sdum
        b2g1[pl.ds(c21 + 16 * t, 16)] = gdum
        b2s1[pl.ds(c21 + 16 * t, 16)] = sdum

    cnt_v[0] = jnp.full((16,), 1, jnp.int32) * c10
    cnt_v[1] = jnp.full((16,), 1, jnp.int32) * c11
    cnt_v[2] = jnp.full((16,), 1, jnp.int32) * c20
    cnt_v[3] = jnp.full((16,), 1, jnp.int32) * c21

    pltpu.sync_copy(dN_v, dN_out.at[wid])
    pltpu.sync_copy(dE_v, dE_out.at[wid])
    pltpu.sync_copy(b1g0, b1g_o.at[wid].at[0])
    pltpu.sync_copy(b1g1, b1g_o.at[wid].at[1])
    pltpu.sync_copy(b1s0, b1s_o.at[wid].at[0])
    pltpu.sync_copy(b1s1, b1s_o.at[wid].at[1])
    pltpu.sync_copy(b2g0, b2g_o.at[wid].at[0])
    pltpu.sync_copy(b2g1, b2g_o.at[wid].at[1])
    pltpu.sync_copy(b2s0, b2s_o.at[wid].at[0])
    pltpu.sync_copy(b2s1, b2s_o.at[wid].at[1])
    pltpu.sync_copy(cnt_v, cnt_o.at[wid])


_sc_scan = pl.kernel(
    _sc_scan_body,
    out_type=[jax.ShapeDtypeStruct((NW, NP), jnp.float32),
              jax.ShapeDtypeStruct((NW, NP), jnp.float32),
              jax.ShapeDtypeStruct((NW, 2, CAP), jnp.int32),
              jax.ShapeDtypeStruct((NW, 2, CAP), jnp.int32),
              jax.ShapeDtypeStruct((NW, 2, CAP), jnp.int32),
              jax.ShapeDtypeStruct((NW, 2, CAP), jnp.int32),
              jax.ShapeDtypeStruct((NW, 4, 16), jnp.int32)],
    mesh=_MESH,
    scratch_types=[
        pltpu.VMEM((CHUNK,), jnp.int32),
        pltpu.VMEM((CHUNK,), jnp.int32),
        pltpu.VMEM((NP,), jnp.float32),
        pltpu.VMEM((NP,), jnp.float32),
        pltpu.VMEM((CAP,), jnp.int32),
        pltpu.VMEM((CAP,), jnp.int32),
        pltpu.VMEM((CAP,), jnp.int32),
        pltpu.VMEM((CAP,), jnp.int32),
        pltpu.VMEM((CAP,), jnp.int32),
        pltpu.VMEM((CAP,), jnp.int32),
        pltpu.VMEM((CAP,), jnp.int32),
        pltpu.VMEM((CAP,), jnp.int32),
        pltpu.VMEM((4, 16), jnp.int32),
    ],
    compiler_params=pltpu.CompilerParams(needs_layout_passes=False),
)


def _sc_scatter_body(tab, bg, bs, cnt2, s_out, rows_a, rows_b, ig0, is0,
                     cnt_v, tab_sh, acc_sh, gsem_a, gsem_b, isem0, isem1):
    c = lax.axis_index("c")
    s = lax.axis_index("s")

    tb = pl.multiple_of(jnp.minimum(s * 632, TROW - 632), 8)
    pltpu.sync_copy(tab.at[pl.ds(tb, 632)], tab_sh.at[pl.ds(tb, 632)])

    def zrow(i, carry):
        def zlane(j, carry2):
            rows_a[i, pl.ds(j * 16, 16)] = jnp.zeros((16,), jnp.float32)
            return carry2
        return lax.fori_loop(0, D // 16, zlane, carry)
    lax.fori_loop(0, BLK, zrow, 0)
    zb = pl.multiple_of(jnp.minimum(s * 328, ACC - 328), 8)
    for q in range(10):
        pltpu.sync_copy(rows_a,
                        acc_sh.at[pl.ds(pl.multiple_of(zb + q * 32, 8), 32)])
    pltpu.sync_copy(rows_a.at[pl.ds(0, 8)],
                    acc_sh.at[pl.ds(pl.multiple_of(zb + 320, 8), 8)])
    plsc.subcore_barrier()

    for li in range(2):
        w = 2 * s + li
        pltpu.sync_copy(cnt2.at[w].at[c], cnt_v)
        k = jnp.max(cnt_v[...])
        nch = (k + 127) // 128

        def chunk(q, carry):
            qo = pl.multiple_of(q * 4, 4)
            cg = pltpu.async_copy(bg.at[w].at[c].at[pl.ds(qo, 4)], ig0,
                                  isem0)
            cs = pltpu.async_copy(bs.at[w].at[c].at[pl.ds(qo, 4)], is0,
                                  isem1)
            cg.wait()
            cs.wait()
            pltpu.async_copy(tab_sh.at[ig0.at[0]], rows_a, gsem_a)
            for b in range(0, 4, 2):
                pltpu.make_async_copy(tab_sh.at[ig0.at[b]], rows_a,
                                      gsem_a).wait()
                pltpu.async_copy(tab_sh.at[ig0.at[b + 1]], rows_b, gsem_b)
                pltpu.sync_copy(rows_a, acc_sh.at[is0.at[b]], add=True)
                pltpu.make_async_copy(tab_sh.at[ig0.at[b + 1]], rows_b,
                                      gsem_b).wait()
                if b + 2 < 4:
                    pltpu.async_copy(tab_sh.at[ig0.at[b + 2]], rows_a,
                                     gsem_a)
                pltpu.sync_copy(rows_b, acc_sh.at[is0.at[b + 1]], add=True)
            return carry
        lax.fori_loop(0, nch, chunk, 0)
    plsc.subcore_barrier()

    for q in range(10):
        r0 = pl.multiple_of(s * 320 + q * 32, 8)
        pltpu.sync_copy(acc_sh.at[pl.ds(r0, 32)], rows_a)
        pltpu.sync_copy(
            rows_a, s_out.at[pl.ds(pl.multiple_of(c * HALF + r0, 8), 32)])


_sc_scatter = pl.kernel(
    _sc_scatter_body,
    out_type=[jax.ShapeDtypeStruct((NP, D), jnp.float32)],
    mesh=_MESH,
    scratch_types=[
        pltpu.VMEM((BLK, D), jnp.float32),
        pltpu.VMEM((BLK, D), jnp.float32),
        pltpu.VMEM((4, BLK), jnp.int32),
        pltpu.VMEM((4, BLK), jnp.int32),
        pltpu.VMEM((16,), jnp.int32),
        pltpu.VMEM_SHARED((TROW, D), jnp.float32),
        pltpu.VMEM_SHARED((ACC, D), jnp.float32),
        pltpu.SemaphoreType.DMA,
        pltpu.SemaphoreType.DMA,
        pltpu.SemaphoreType.DMA,
        pltpu.SemaphoreType.DMA,
    ],
    compiler_params=pltpu.CompilerParams(needs_layout_passes=False),
)


def kernel(x, H, theta, bias):
    x = x.astype(jnp.float32)
    nidx = H[0].astype(jnp.int32).reshape(NW, CHUNK)
    eidx = H[1].astype(jnp.int32).reshape(NW, CHUNK)
    x_pad = jnp.concatenate([x, jnp.zeros((NP - N, D), jnp.float32)])

    dN_p, dE_p, b1g, b1s, b2g, b2s, cnt = _sc_scan(nidx, eidx)
    b1g = b1g.reshape(NW, 2, CAP // BLK, BLK)
    b1s = b1s.reshape(NW, 2, CAP // BLK, BLK)
    b2g = b2g.reshape(NW, 2, CAP // BLK, BLK)
    b2s = b2s.reshape(NW, 2, CAP // BLK, BLK)

    xw = _matmul(x_pad, theta)
    (s1,) = _sc_scatter(xw[:TROW], b1g, b1s, cnt[:, 0:2])
    zeros_row = jnp.zeros((1, D), jnp.float32)
    hyedge_ft = _combine(s1, dE_p, zeros_row)
    (s2,) = _sc_scatter(hyedge_ft[:TROW], b2g, b2s, cnt[:, 2:4])
    node_ft = _combine(s2, dN_p, bias.reshape(1, D))
    return node_ft[:N]
